# baseline (speedup 1.0000x reference)
import functools

import jax
import jax.numpy as jnp
from jax import lax
from jax.experimental import pallas as pl
from jax.experimental.pallas import tpu as pltpu, tpu_sc as plsc

NC = 2
NS = 16
NW = NC * NS
LANES = 16

SLAB = 8
UC = 2048
NB = 2


def _sc_bank_update(sp, mp, np_, nmp, sid, perm, srcw, cs, n_users, batch, d):
    nfull = (n_users // UC) // NB * NB
    tails = []
    off = nfull * UC
    while off < n_users:
        sz = min(UC, n_users - off)
        tails.append((off, sz))
        off += sz
    tail_last = tails[-1][1] if tails else 0
    ncs = cs.shape[0]

    mesh = plsc.VectorSubcoreMesh(core_axis_name="c", subcore_axis_name="s",
                                  num_cores=NC, num_subcores=NS)

    @functools.partial(
        pl.kernel,
        mesh=mesh,
        out_type=(
            jax.ShapeDtypeStruct((d, batch), jnp.float32),
            jax.ShapeDtypeStruct((d, batch), jnp.float32),
            jax.ShapeDtypeStruct((d, n_users), jnp.float32),
            jax.ShapeDtypeStruct((d, n_users), jnp.float32),
        ),
        scratch_types=[
            pltpu.VMEM((SLAB, UC), jnp.float32),
            pltpu.VMEM((SLAB, UC), jnp.float32),
            pltpu.VMEM((SLAB, batch), jnp.float32),
            pltpu.VMEM((SLAB, batch), jnp.float32),
            pltpu.VMEM((batch,), jnp.int32),
            pltpu.VMEM((batch,), jnp.int32),
            pltpu.VMEM((batch,), jnp.int32),
            pltpu.VMEM((ncs,), jnp.int32),
            pltpu.VMEM((SLAB, max(tail_last, 8)), jnp.float32),
            pltpu.SemaphoreType.DMA,
            pltpu.SemaphoreType.DMA,
            pltpu.SemaphoreType.DMA,
            pltpu.SemaphoreType.DMA,
            pltpu.SemaphoreType.DMA,
            pltpu.SemaphoreType.DMA,
            pltpu.SemaphoreType.DMA,
        ],
        compiler_params=pltpu.CompilerParams(needs_layout_passes=False),
    )
    def k(s_hbm, m_hbm, n_hbm, nm_hbm, sid_hbm, perm_hbm, src_hbm, cs_hbm,
          bs_hbm, bm_hbm, us_hbm, um_hbm,
          rbuf0, rbuf1, pbuf, nbuf, sidv, permv, srcv, csv, tailbuf,
          si0, si1, si2, so0, so1, so2, sem):
        ring = (rbuf0, rbuf1)
        wid = lax.axis_index("s") * NC + lax.axis_index("c")
        si = (si0, si1, si2)
        so = (so0, so1, so2)
        lanes = lax.iota(jnp.int32, LANES)
        fvecs = [jnp.full((LANES,), f, jnp.int32) for f in range(SLAB)]

        pltpu.sync_copy(sid_hbm, sidv)
        pltpu.sync_copy(perm_hbm, permv)
        pltpu.sync_copy(src_hbm, srcv)
        pltpu.sync_copy(cs_hbm, csv)

        def span(c):
            v = jnp.full((LANES,), c, jnp.int32)
            s = jnp.max(plsc.load_gather(csv, [v]))
            e = jnp.max(plsc.load_gather(csv, [v + 1]))
            return s, e

        def process_chunk(cbuf, c, u0):
            s, e = span(c)
            trips = (e - s + (LANES - 1)) // LANES

            def gbody(t, carry):
                pos = jnp.minimum(s + t * LANES + lanes, e - 1)
                iu = plsc.load_gather(sidv, [pos]) - u0
                ib = plsc.load_gather(permv, [pos])
                for f in range(SLAB):
                    vals = plsc.load_gather(cbuf, [fvecs[f], iu])
                    plsc.store_scatter(pbuf, [fvecs[f], ib], vals)
                return carry

            lax.fori_loop(0, trips, gbody, 0)

            def sbody(t, carry):
                pos = jnp.minimum(s + t * LANES + lanes, e - 1)
                iu = plsc.load_gather(sidv, [pos]) - u0
                isr = plsc.load_gather(srcv, [pos])
                for f in range(SLAB):
                    nv = plsc.load_gather(nbuf, [fvecs[f], isr])
                    plsc.store_scatter(cbuf, [fvecs[f], iu], nv)
                return carry

            lax.fori_loop(0, trips, sbody, 0)

        def do_slab(src_hbm_, new_hbm_, out_hbm_, bout_hbm_, srow):
            pltpu.sync_copy(new_hbm_.at[pl.ds(srow, SLAB)], nbuf)

            def tri(i, carry):
                for b in range(NB):
                    c = i * NB + b

                    @pl.when(i > 0)
                    def _drain():
                        pltpu.make_async_copy(
                            ring[b],
                            out_hbm_.at[pl.ds(0, SLAB), pl.ds(0, UC)],
                            so[b]).wait()

                    u0 = pl.multiple_of(c * UC, 128)
                    pltpu.async_copy(
                        src_hbm_.at[pl.ds(srow, SLAB), pl.ds(u0, UC)],
                        ring[b], si[b])
                for b in range(NB):
                    c = i * NB + b
                    u0 = pl.multiple_of(c * UC, 128)
                    pltpu.make_async_copy(
                        src_hbm_.at[pl.ds(0, SLAB), pl.ds(0, UC)],
                        ring[b], si[b]).wait()
                    process_chunk(ring[b], c, u0)
                    pltpu.async_copy(
                        ring[b],
                        out_hbm_.at[pl.ds(srow, SLAB), pl.ds(u0, UC)],
                        so[b])
                return carry

            lax.fori_loop(0, nfull // NB, tri, 0)
            for b in range(NB):
                pltpu.make_async_copy(
                    ring[b],
                    out_hbm_.at[pl.ds(0, SLAB), pl.ds(0, UC)],
                    so[b]).wait()

            for t, (toff, tsz) in enumerate(tails):
                tbuf = ring[t] if tsz == UC else tailbuf
                pltpu.async_copy(
                    src_hbm_.at[pl.ds(srow, SLAB), pl.ds(toff, tsz)],
                    tbuf, sem).wait()
                process_chunk(tbuf, nfull + t, toff)
                pltpu.async_copy(
                    tbuf,
                    out_hbm_.at[pl.ds(srow, SLAB), pl.ds(toff, tsz)],
                    sem).wait()

            pltpu.sync_copy(pbuf, bout_hbm_.at[pl.ds(srow, SLAB)])

        for (sh, nh, oh, bh) in ((s_hbm, n_hbm, us_hbm, bs_hbm),
                                 (m_hbm, nm_hbm, um_hbm, bm_hbm)):
            for soff in range(0, d // NW, SLAB):
                do_slab(sh, nh, oh, bh,
                        pl.multiple_of(wid * (d // NW) + soff, 8))

    return k(sp, mp, np_, nmp, sid, perm, srcw, cs)


def kernel(user_ids, new_states, new_momentums, states, momentums):
    n_users = states.shape[0]
    batch = user_ids.shape[0]
    trail = states.shape[1:]
    d = 1
    for t in trail:
        d *= t

    def phys(x):
        return x.transpose(1, 2, 3, 0).reshape(d, x.shape[0])

    sp, mp = phys(states), phys(momentums)
    np_, nmp = phys(new_states), phys(new_momentums)
    uid = user_ids.astype(jnp.int32)

    sid, perm = lax.sort(
        (uid, jnp.arange(batch, dtype=jnp.int32)), num_keys=1, is_stable=True)
    is_last = jnp.concatenate(
        [sid[1:] != sid[:-1], jnp.ones((1,), jnp.bool_)])
    cand = jnp.where(is_last, jnp.arange(batch, dtype=jnp.int32), batch)
    last_pos = lax.cummin(cand[::-1])[::-1]
    srcw = perm[last_pos]

    nchunk = -(-n_users // UC)
    bucket = sid // UC
    cnts = jnp.sum((bucket[None, :] ==
                    jnp.arange(nchunk, dtype=jnp.int32)[:, None])
                   .astype(jnp.int32), axis=1)
    cs = jnp.concatenate([jnp.zeros((1,), jnp.int32),
                          jnp.cumsum(cnts, dtype=jnp.int32)])
    pad = (-(nchunk + 1)) % 8
    cs = jnp.concatenate([cs, jnp.full((pad,), batch, jnp.int32)])

    bs_p, bm_p, us_p, um_p = _sc_bank_update(
        sp, mp, np_, nmp, sid, perm, srcw, cs, n_users, batch, d)

    def unphys(x, n):
        return x.reshape(trail + (n,)).transpose(3, 0, 1, 2)

    return (unphys(bs_p, batch), unphys(bm_p, batch),
            unphys(us_p, n_users), unphys(um_p, n_users))


# speedup vs baseline: 1.0619x; 1.0619x over previous
import functools

import jax
import jax.numpy as jnp
from jax import lax
from jax.experimental import pallas as pl
from jax.experimental.pallas import tpu as pltpu, tpu_sc as plsc

NC = 2
NS = 16
NW = NC * NS
LANES = 16

SLAB = 8
UC = 1664


def _sc_bank_update(sp, mp, np_, nmp, sid, perm, srcw, cs, n_users, batch, d):
    nfull = (n_users // UC) // 3 * 3
    tails = []
    off = nfull * UC
    while off < n_users:
        sz = min(UC, n_users - off)
        tails.append((off, sz))
        off += sz
    tail_last = tails[-1][1] if tails else 0
    ncs = cs.shape[0]

    mesh = plsc.VectorSubcoreMesh(core_axis_name="c", subcore_axis_name="s",
                                  num_cores=NC, num_subcores=NS)

    @functools.partial(
        pl.kernel,
        mesh=mesh,
        out_type=(
            jax.ShapeDtypeStruct((d, batch), jnp.float32),
            jax.ShapeDtypeStruct((d, batch), jnp.float32),
            jax.ShapeDtypeStruct((d, n_users), jnp.float32),
            jax.ShapeDtypeStruct((d, n_users), jnp.float32),
        ),
        scratch_types=[
            pltpu.VMEM((SLAB, UC), jnp.float32),
            pltpu.VMEM((SLAB, UC), jnp.float32),
            pltpu.VMEM((SLAB, UC), jnp.float32),
            pltpu.VMEM((SLAB, batch), jnp.float32),
            pltpu.VMEM((SLAB, batch), jnp.float32),
            pltpu.VMEM((batch,), jnp.int32),
            pltpu.VMEM((batch,), jnp.int32),
            pltpu.VMEM((batch,), jnp.int32),
            pltpu.VMEM((ncs,), jnp.int32),
            pltpu.VMEM((SLAB, max(tail_last, 8)), jnp.float32),
            pltpu.SemaphoreType.DMA,
            pltpu.SemaphoreType.DMA,
            pltpu.SemaphoreType.DMA,
            pltpu.SemaphoreType.DMA,
            pltpu.SemaphoreType.DMA,
            pltpu.SemaphoreType.DMA,
            pltpu.SemaphoreType.DMA,
        ],
        compiler_params=pltpu.CompilerParams(needs_layout_passes=False),
    )
    def k(s_hbm, m_hbm, n_hbm, nm_hbm, sid_hbm, perm_hbm, src_hbm, cs_hbm,
          bs_hbm, bm_hbm, us_hbm, um_hbm,
          rbuf0, rbuf1, rbuf2, pbuf, nbuf, sidv, permv, srcv, csv, tailbuf,
          si0, si1, si2, so0, so1, so2, sem):
        ring = (rbuf0, rbuf1, rbuf2)
        wid = lax.axis_index("s") * NC + lax.axis_index("c")
        si = (si0, si1, si2)
        so = (so0, so1, so2)
        lanes = lax.iota(jnp.int32, LANES)
        fvecs = [jnp.full((LANES,), f, jnp.int32) for f in range(SLAB)]

        pltpu.sync_copy(sid_hbm, sidv)
        pltpu.sync_copy(perm_hbm, permv)
        pltpu.sync_copy(src_hbm, srcv)
        pltpu.sync_copy(cs_hbm, csv)

        def span(c):
            v = jnp.full((LANES,), c, jnp.int32)
            s = jnp.max(plsc.load_gather(csv, [v]))
            e = jnp.max(plsc.load_gather(csv, [v + 1]))
            return s, e

        def process_chunk(cbuf, c, u0):
            s, e = span(c)
            trips = (e - s + (LANES - 1)) // LANES

            def gbody(t, carry):
                pos = jnp.minimum(s + t * LANES + lanes, e - 1)
                iu = plsc.load_gather(sidv, [pos]) - u0
                ib = plsc.load_gather(permv, [pos])
                for f in range(SLAB):
                    vals = plsc.load_gather(cbuf, [fvecs[f], iu])
                    plsc.store_scatter(pbuf, [fvecs[f], ib], vals)
                return carry

            lax.fori_loop(0, trips, gbody, 0)

            def sbody(t, carry):
                pos = jnp.minimum(s + t * LANES + lanes, e - 1)
                iu = plsc.load_gather(sidv, [pos]) - u0
                isr = plsc.load_gather(srcv, [pos])
                for f in range(SLAB):
                    nv = plsc.load_gather(nbuf, [fvecs[f], isr])
                    plsc.store_scatter(cbuf, [fvecs[f], iu], nv)
                return carry

            lax.fori_loop(0, trips, sbody, 0)

        def do_slab(src_hbm_, new_hbm_, out_hbm_, bout_hbm_, srow):
            pltpu.sync_copy(new_hbm_.at[pl.ds(srow, SLAB)], nbuf)

            def tri(i, carry):
                for b in range(3):
                    c = i * 3 + b

                    @pl.when(i > 0)
                    def _drain():
                        pltpu.make_async_copy(
                            ring[b],
                            out_hbm_.at[pl.ds(0, SLAB), pl.ds(0, UC)],
                            so[b]).wait()

                    u0 = pl.multiple_of(c * UC, 128)
                    pltpu.async_copy(
                        src_hbm_.at[pl.ds(srow, SLAB), pl.ds(u0, UC)],
                        ring[b], si[b])
                for b in range(3):
                    c = i * 3 + b
                    u0 = pl.multiple_of(c * UC, 128)
                    pltpu.make_async_copy(
                        src_hbm_.at[pl.ds(0, SLAB), pl.ds(0, UC)],
                        ring[b], si[b]).wait()
                    process_chunk(ring[b], c, u0)
                    pltpu.async_copy(
                        ring[b],
                        out_hbm_.at[pl.ds(srow, SLAB), pl.ds(u0, UC)],
                        so[b])
                return carry

            lax.fori_loop(0, nfull // 3, tri, 0)
            for b in range(3):
                pltpu.make_async_copy(
                    ring[b],
                    out_hbm_.at[pl.ds(0, SLAB), pl.ds(0, UC)],
                    so[b]).wait()

            for t, (toff, tsz) in enumerate(tails):
                tbuf = ring[t] if tsz == UC else tailbuf
                pltpu.async_copy(
                    src_hbm_.at[pl.ds(srow, SLAB), pl.ds(toff, tsz)],
                    tbuf, sem).wait()
                process_chunk(tbuf, nfull + t, toff)
                pltpu.async_copy(
                    tbuf,
                    out_hbm_.at[pl.ds(srow, SLAB), pl.ds(toff, tsz)],
                    sem).wait()

            pltpu.sync_copy(pbuf, bout_hbm_.at[pl.ds(srow, SLAB)])

        for (sh, nh, oh, bh) in ((s_hbm, n_hbm, us_hbm, bs_hbm),
                                 (m_hbm, nm_hbm, um_hbm, bm_hbm)):
            for soff in range(0, d // NW, SLAB):
                do_slab(sh, nh, oh, bh,
                        pl.multiple_of(wid * (d // NW) + soff, 8))

    return k(sp, mp, np_, nmp, sid, perm, srcw, cs)


def kernel(user_ids, new_states, new_momentums, states, momentums):
    n_users = states.shape[0]
    batch = user_ids.shape[0]
    trail = states.shape[1:]
    d = 1
    for t in trail:
        d *= t

    def phys(x):
        return x.transpose(1, 2, 3, 0).reshape(d, x.shape[0])

    sp, mp = phys(states), phys(momentums)
    np_, nmp = phys(new_states), phys(new_momentums)
    uid = user_ids.astype(jnp.int32)

    sid, perm = lax.sort(
        (uid, jnp.arange(batch, dtype=jnp.int32)), num_keys=1, is_stable=True)
    is_last = jnp.concatenate(
        [sid[1:] != sid[:-1], jnp.ones((1,), jnp.bool_)])
    cand = jnp.where(is_last, jnp.arange(batch, dtype=jnp.int32), batch)
    last_pos = lax.cummin(cand[::-1])[::-1]
    srcw = perm[last_pos]

    nchunk = -(-n_users // UC)
    bucket = sid // UC
    cnts = jnp.sum((bucket[None, :] ==
                    jnp.arange(nchunk, dtype=jnp.int32)[:, None])
                   .astype(jnp.int32), axis=1)
    cs = jnp.concatenate([jnp.zeros((1,), jnp.int32),
                          jnp.cumsum(cnts, dtype=jnp.int32)])
    pad = (-(nchunk + 1)) % 8
    cs = jnp.concatenate([cs, jnp.full((pad,), batch, jnp.int32)])

    bs_p, bm_p, us_p, um_p = _sc_bank_update(
        sp, mp, np_, nmp, sid, perm, srcw, cs, n_users, batch, d)

    def unphys(x, n):
        return x.reshape(trail + (n,)).transpose(3, 0, 1, 2)

    return (unphys(bs_p, batch), unphys(bm_p, batch),
            unphys(us_p, n_users), unphys(um_p, n_users))
